# baseline (speedup 1.0000x reference)
import jax
import jax.numpy as jnp
from jax import lax
from jax.experimental import pallas as pl
from jax.experimental.pallas import tpu as pltpu

_XCHUNKS = 16


def _body(id_ref, x_hbm, w_hbm, out_hbm, wbuf, slab, sem_x, sem_r, sem_s):
    B, C, SH, H = x_hbm.shape
    D = w_hbm.shape[1]
    S = SH // H

    nb = B // _XCHUNKS
    for k in range(_XCHUNKS):
        pltpu.make_async_copy(
            x_hbm.at[pl.ds(k * nb, nb)],
            out_hbm.at[pl.ds(k * nb, nb), pl.ds(0, C)],
            sem_x.at[k % 2],
        ).start()

    def row_fetch(b, p):
        grp = 8 * (id_ref[b] // 8)
        return pltpu.make_async_copy(
            w_hbm.at[pl.ds(grp, 8), :], wbuf.at[p], sem_r.at[p]
        )

    def slab_copy(b, p):
        return pltpu.make_async_copy(slab.at[p], out_hbm.at[b, C], sem_s.at[p])

    row_fetch(0, 0).start()

    def step(b, carry):
        p = lax.rem(b, 2)

        @pl.when(b + 1 < B)
        def _():
            row_fetch(b + 1, 1 - p).start()

        row_fetch(b, p).wait()
        row = id_ref[b] % 8
        w64 = wbuf[p, pl.ds(row, 1), :].reshape(H, H)

        @pl.when(b >= 2)
        def _():
            slab_copy(b - 2, p).wait()

        for t in range(S):
            slab[p, pl.ds(H * t, H), :] = w64
        slab_copy(b, p).start()
        return carry

    lax.fori_loop(0, B, step, 0)

    slab_copy(B - 2, 0).wait()
    slab_copy(B - 1, 1).wait()
    for k in range(_XCHUNKS):
        pltpu.make_async_copy(
            x_hbm.at[pl.ds(k * nb, nb)],
            out_hbm.at[pl.ds(k * nb, nb), pl.ds(0, C)],
            sem_x.at[k % 2],
        ).wait()


def kernel(x, id, W):
    b, c, s, h, _ = x.shape
    sh = s * h
    x4 = x.reshape(b, c, sh, h)
    out = pl.pallas_call(
        _body,
        grid=(1,),
        in_specs=[
            pl.BlockSpec(memory_space=pltpu.SMEM),
            pl.BlockSpec(memory_space=pltpu.MemorySpace.HBM),
            pl.BlockSpec(memory_space=pltpu.MemorySpace.HBM),
        ],
        out_specs=pl.BlockSpec(memory_space=pltpu.MemorySpace.HBM),
        out_shape=jax.ShapeDtypeStruct((b, c + 1, sh, h), x.dtype),
        scratch_shapes=[
            pltpu.VMEM((2, 8, h * h), jnp.float32),
            pltpu.VMEM((2, sh, h), jnp.float32),
            pltpu.SemaphoreType.DMA((2,)),
            pltpu.SemaphoreType.DMA((2,)),
            pltpu.SemaphoreType.DMA((2,)),
        ],
    )(id, x4, W)
    return out.reshape(b, c + 1, s, h, h)


# speedup vs baseline: 11.4510x; 11.4510x over previous
import jax
import jax.numpy as jnp
from jax.experimental import pallas as pl
from jax.experimental.pallas import tpu as pltpu


def _body(id_ref, x_ref, w_ref, out_ref):
    i = pl.program_id(0)
    c = x_ref.shape[1]
    s = x_ref.shape[2]
    h = x_ref.shape[3]
    out_ref[0, :c] = x_ref[0]
    row = id_ref[i] % w_ref.shape[0]
    w64 = w_ref[pl.ds(row, 1), :].reshape(h, h)
    for t in range(s):
        out_ref[0, c, t] = w64


def kernel(x, id, W):
    b, c, s, h, _ = x.shape
    grid_spec = pltpu.PrefetchScalarGridSpec(
        num_scalar_prefetch=1,
        grid=(b,),
        in_specs=[
            pl.BlockSpec((1, c, s, h, h), lambda i, idr: (i, 0, 0, 0, 0)),
            pl.BlockSpec((8, h * h), lambda i, idr: (idr[i] // 8, 0)),
        ],
        out_specs=pl.BlockSpec((1, c + 1, s, h, h), lambda i, idr: (i, 0, 0, 0, 0)),
    )
    return pl.pallas_call(
        _body,
        grid_spec=grid_spec,
        out_shape=jax.ShapeDtypeStruct((b, c + 1, s, h, h), x.dtype),
    )(id, x, W)
